# baseline (speedup 1.0000x reference)
import functools

import jax
import jax.numpy as jnp
from jax import lax
from jax.experimental import pallas as pl
from jax.experimental.pallas import tpu as pltpu
from jax.experimental.pallas import tpu_sc as plsc

FEATURE_NUM = 26
VOCAB = 100000
EMBED_DIM = 32
BATCH = 16384

_L = 16
_NC = 2
_NS = 16
_NW = _NC * _NS

_ROWS = BATCH * FEATURE_NUM
_NTILE = FEATURE_NUM * VOCAB // 8
_NQ = FEATURE_NUM * VOCAB // 4

_BV = 256
_BPF = 416
_P1_PW = FEATURE_NUM * _BPF // _NW

_RPW = _ROWS // _NW
_CH = 64
_NCH = _RPW // _CH
_OPC = _CH * EMBED_DIM // 128
_OPW = _RPW * EMBED_DIM // 128


def _p1_body(table_hbm, stg_hbm, vb0, vb1, vp0, vp1, rsem0, rsem1, wsem0, wsem1):
    wid = lax.axis_index("s") * _NC + lax.axis_index("c")

    def fv_of(i):
        blk = wid + i * _NW
        f = lax.div(blk, _BPF)
        v0 = lax.min(lax.rem(blk, _BPF) * _BV, VOCAB - _BV)
        return f, v0

    def fire_read(i, vb, rsem):
        f, v0 = fv_of(i)
        pltpu.make_async_copy(
            table_hbm.at[f, pl.ds(v0, _BV)], vb, rsem
        ).start()

    def repack(vb, vp):
        def one(q, _):
            for qq in range(4):
                for h in range(4):
                    for t in range(2):
                        vp[4 * q + qq, pl.ds(h * EMBED_DIM + t * _L, _L)] = (
                            vb[16 * q + 4 * qq + h, pl.ds(t * _L, _L)]
                        )
            return 0

        lax.fori_loop(0, _BV // 16, one, 0)

    def wb(i, vp, wsem):
        f, v0 = fv_of(i)
        q0 = 2 * (f * (VOCAB // 8) + lax.div(v0, 8))
        pltpu.make_async_copy(
            vp, stg_hbm.at[pl.ds(q0, 2 * _BV // 8)], wsem
        ).start()

    def wb_wait(vp, wsem):
        pltpu.make_async_copy(
            vp, stg_hbm.at[pl.ds(0, 2 * _BV // 8)], wsem
        ).wait()

    vbs = (vb0, vb1)
    vps = (vp0, vp1)
    rsems = (rsem0, rsem1)
    wsems = (wsem0, wsem1)

    fire_read(0, vb0, rsem0)
    fire_read(1, vb1, rsem1)

    def step(kk, _):
        for par in range(2):
            i = 2 * kk + par
            f, v0 = fv_of(i)
            pltpu.make_async_copy(
                table_hbm.at[f, pl.ds(v0, _BV)], vbs[par], rsems[par]
            ).wait()

            @pl.when(kk >= 1)
            def _wait_wb():
                wb_wait(vps[par], wsems[par])

            repack(vbs[par], vps[par])

            @pl.when(i + 2 < _P1_PW)
            def _next_read():
                fire_read(i + 2, vbs[par], rsems[par])

            wb(i, vps[par], wsems[par])
        return 0

    lax.fori_loop(0, _P1_PW // 2, step, 0)

    for par in range(2):
        wb_wait(vps[par], wsems[par])


def _p2_body(x_hbm, stg_hbm, out_hbm, qv, sv, g0, g1, g2, g3, ob0, ob1,
             gsem0, gsem1, gsem2, gsem3, wsem0, wsem1):
    wid = lax.axis_index("s") * _NC + lax.axis_index("c")
    xrow0 = wid * (_RPW // 128)
    orow0 = wid * _OPW

    pltpu.sync_copy(x_hbm.at[pl.ds(xrow0, _RPW // 128)], qv)

    lanes = lax.iota(jnp.int32, _L)

    def compute_row(j2, _):
        for t in range(128 // _L):
            r = j2 * 128 + t * _L + lanes
            f = lax.rem(r, FEATURE_NUM)
            flat = qv[j2, pl.ds(t * _L, _L)] + f * VOCAB
            qv[j2, pl.ds(t * _L, _L)] = lax.shift_right_logical(flat, 2)
            sv[j2, pl.ds(t * _L, _L)] = lax.bitwise_and(flat, 3)
        return 0

    lax.fori_loop(0, _RPW // 128, compute_row, 0)

    gbufs = (g0, g1, g2, g3)
    gsems = (gsem0, gsem1, gsem2, gsem3)
    obufs = (ob0, ob1)
    wsems = (wsem0, wsem1)

    def idx_of(jj, m):
        return qv.at[2 * jj + m // 2, pl.ds((m % 2) * _CH, _CH)]

    def fire(jj, m, gbuf, gsem):
        pltpu.make_async_copy(stg_hbm.at[idx_of(jj, m)], gbuf, gsem).start()

    def extract(jj, m, gbuf, ob):
        for g in range(_CH // _L):
            pvec = g * _L + lanes
            svec = sv[2 * jj + m // 2, pl.ds((m % 2) * _CH + g * _L, _L)]
            gbase = svec * EMBED_DIM
            orow = lax.shift_right_logical(pvec, 2)
            ocol0 = lax.bitwise_and(pvec, 3) * EMBED_DIM
            for c in range(EMBED_DIM):
                cc = jnp.full((_L,), c, jnp.int32)
                v = plsc.load_gather(gbuf, [pvec, gbase + cc])
                plsc.store_scatter(ob, [orow, ocol0 + cc], v)

    for m in range(4):
        fire(0, m, gbufs[m], gsems[m])

    def step(jj, _):
        for m in range(4):
            j = 4 * jj + m
            pltpu.make_async_copy(
                stg_hbm.at[idx_of(jj, m)], gbufs[m], gsems[m]
            ).wait()

            @pl.when(j >= 2)
            def _wait_wb():
                pltpu.make_async_copy(
                    obufs[m % 2], out_hbm.at[pl.ds(orow0, _OPC)], wsems[m % 2]
                ).wait()

            extract(jj, m, gbufs[m], obufs[m % 2])

            @pl.when(j + 4 < _NCH)
            def _next():
                fire(jj + 1, m, gbufs[m], gsems[m])

            pltpu.make_async_copy(
                obufs[m % 2],
                out_hbm.at[pl.ds(orow0 + j * _OPC, _OPC)],
                wsems[m % 2],
            ).start()
        return 0

    lax.fori_loop(0, _NCH // 4, step, 0)

    for par in range(2):
        pltpu.make_async_copy(
            obufs[par], out_hbm.at[pl.ds(orow0, _OPC)], wsems[par]
        ).wait()


@jax.jit
def _run(x2d, tables):
    p1 = pl.kernel(
        _p1_body,
        mesh=plsc.VectorSubcoreMesh(core_axis_name="c", subcore_axis_name="s"),
        out_type=jax.ShapeDtypeStruct((_NQ, 128), jnp.float32),
        scratch_types=[
            pltpu.VMEM((_BV, EMBED_DIM), jnp.float32),
            pltpu.VMEM((_BV, EMBED_DIM), jnp.float32),
            pltpu.VMEM((2 * _BV // 8, 128), jnp.float32),
            pltpu.VMEM((2 * _BV // 8, 128), jnp.float32),
            pltpu.SemaphoreType.DMA,
            pltpu.SemaphoreType.DMA,
            pltpu.SemaphoreType.DMA,
            pltpu.SemaphoreType.DMA,
        ],
        compiler_params=pltpu.CompilerParams(
            use_tc_tiling_on_sc=True, needs_layout_passes=False
        ),
    )
    stg = p1(tables)

    p2 = pl.kernel(
        _p2_body,
        mesh=plsc.VectorSubcoreMesh(core_axis_name="c", subcore_axis_name="s"),
        out_type=jax.ShapeDtypeStruct((_ROWS * EMBED_DIM // 128, 128), jnp.float32),
        scratch_types=[
            pltpu.VMEM((_RPW // 128, 128), jnp.int32),
            pltpu.VMEM((_RPW // 128, 128), jnp.int32),
            pltpu.VMEM((_CH, 128), jnp.float32),
            pltpu.VMEM((_CH, 128), jnp.float32),
            pltpu.VMEM((_CH, 128), jnp.float32),
            pltpu.VMEM((_CH, 128), jnp.float32),
            pltpu.VMEM((_OPC, 128), jnp.float32),
            pltpu.VMEM((_OPC, 128), jnp.float32),
            pltpu.SemaphoreType.DMA,
            pltpu.SemaphoreType.DMA,
            pltpu.SemaphoreType.DMA,
            pltpu.SemaphoreType.DMA,
            pltpu.SemaphoreType.DMA,
            pltpu.SemaphoreType.DMA,
        ],
        compiler_params=pltpu.CompilerParams(
            use_tc_tiling_on_sc=True, needs_layout_passes=False
        ),
    )
    return p2(x2d, stg)


def kernel(x, tables):
    x2d = x.astype(jnp.int32).reshape(_ROWS // 128, 128)
    out = _run(x2d, tables)
    return out.reshape(BATCH, FEATURE_NUM, EMBED_DIM)


# speedup vs baseline: 1.3196x; 1.3196x over previous
import functools

import jax
import jax.numpy as jnp
from jax import lax
from jax.experimental import pallas as pl
from jax.experimental.pallas import tpu as pltpu
from jax.experimental.pallas import tpu_sc as plsc

FEATURE_NUM = 26
VOCAB = 100000
EMBED_DIM = 32
BATCH = 16384

_L = 16
_NC = 2
_NS = 16
_NW = _NC * _NS

_ROWS = BATCH * FEATURE_NUM
_NTILE = FEATURE_NUM * VOCAB // 8
_NQ = FEATURE_NUM * VOCAB // 4

_QUADS = _NTILE // 4
_QPW = _QUADS // _NW
_TB = 48
_QPB = _TB // 4
_P1_BLOCKS = 212

_RPW = _ROWS // _NW
_CH = 64
_NCH = _RPW // _CH
_OPC = _CH * EMBED_DIM // 128
_OPW = _RPW * EMBED_DIM // 128


def _p1_body(table_hbm, stg_hbm, vb0, vb1, vp0, vp1, rsem0, rsem1, wsem0, wsem1):
    wid = lax.axis_index("s") * _NC + lax.axis_index("c")
    nq = _QPW + jnp.where(wid < 2, 1, 0)
    baseq = wid * _QPW + lax.min(wid, 2)

    def t0_of(k):
        return 4 * (baseq + lax.min(k * _QPB, nq - _QPB))

    def fire_read(k, vb, rsem):
        pltpu.make_async_copy(
            table_hbm.at[pl.ds(t0_of(k), _TB)], vb, rsem
        ).start()

    def repack(vb, vp):
        def one(q, _):
            for qq in range(4):
                for h in range(4):
                    for t in range(2):
                        vp[4 * q + qq, pl.ds(h * EMBED_DIM + t * _L, _L)] = (
                            vb[2 * q + (4 * qq + h) // 8, (4 * qq + h) % 8,
                               pl.ds(t * _L, _L)]
                        )
            return 0

        lax.fori_loop(0, _TB * 8 // 16, one, 0)

    vbs = (vb0, vb1)
    vps = (vp0, vp1)
    rsems = (rsem0, rsem1)
    wsems = (wsem0, wsem1)

    fire_read(0, vb0, rsem0)
    fire_read(1, vb1, rsem1)

    def step(kk, _):
        for par in range(2):
            k = 2 * kk + par
            pltpu.make_async_copy(
                table_hbm.at[pl.ds(t0_of(k), _TB)], vbs[par], rsems[par]
            ).wait()

            @pl.when(kk >= 1)
            def _wait_wb():
                pltpu.make_async_copy(
                    vps[par], stg_hbm.at[pl.ds(0, 2 * _TB)], wsems[par]
                ).wait()

            repack(vbs[par], vps[par])

            @pl.when(k + 2 < _P1_BLOCKS)
            def _next_read():
                fire_read(k + 2, vbs[par], rsems[par])

            pltpu.make_async_copy(
                vps[par], stg_hbm.at[pl.ds(2 * t0_of(k), 2 * _TB)], wsems[par]
            ).start()
        return 0

    lax.fori_loop(0, _P1_BLOCKS // 2, step, 0)

    for par in range(2):
        pltpu.make_async_copy(
            vps[par], stg_hbm.at[pl.ds(0, 2 * _TB)], wsems[par]
        ).wait()


def _p2_body(x_hbm, stg_hbm, out_hbm, qv, sv, g0, g1, ob0, ob1,
             gsem0, gsem1, wsem0, wsem1):
    wid = lax.axis_index("s") * _NC + lax.axis_index("c")
    xrow0 = wid * (_RPW // 128)
    orow0 = wid * _OPW

    pltpu.sync_copy(x_hbm.at[pl.ds(xrow0, _RPW // 128)], qv)

    lanes = lax.iota(jnp.int32, _L)

    def compute_row(j2, _):
        for t in range(128 // _L):
            r = j2 * 128 + t * _L + lanes
            f = lax.rem(r, FEATURE_NUM)
            flat = qv[j2, pl.ds(t * _L, _L)] + f * VOCAB
            qv[j2, pl.ds(t * _L, _L)] = lax.shift_right_logical(flat, 2)
            sv[j2, pl.ds(t * _L, _L)] = lax.bitwise_and(flat, 3)
        return 0

    lax.fori_loop(0, _RPW // 128, compute_row, 0)

    gbufs = (g0, g1)
    gsems = (gsem0, gsem1)
    obufs = (ob0, ob1)
    wsems = (wsem0, wsem1)

    def idx_of(j):
        return qv.at[lax.div(j, 2), pl.ds(lax.rem(j, 2) * _CH, _CH)]

    def fire(j, gbuf, gsem):
        pltpu.make_async_copy(stg_hbm.at[idx_of(j)], gbuf, gsem).start()

    def extract(j, gbuf, ob):
        for g in range(_CH // _L):
            pvec = g * _L + lanes
            svec = sv[lax.div(j, 2), pl.ds(lax.rem(j, 2) * _CH + g * _L, _L)]
            gbase = svec * EMBED_DIM
            orow = lax.shift_right_logical(pvec, 2)
            ocol0 = lax.bitwise_and(pvec, 3) * EMBED_DIM
            for c in range(EMBED_DIM):
                cc = jnp.full((_L,), c, jnp.int32)
                v = plsc.load_gather(gbuf, [pvec, gbase + cc])
                plsc.store_scatter(ob, [orow, ocol0 + cc], v)

    fire(0, g0, gsem0)
    fire(1, g1, gsem1)

    def step(jj, _):
        for par in range(2):
            j = 2 * jj + par
            pltpu.make_async_copy(
                stg_hbm.at[idx_of(j)], gbufs[par], gsems[par]
            ).wait()

            @pl.when(j >= 2)
            def _wait_wb():
                pltpu.make_async_copy(
                    obufs[par], out_hbm.at[pl.ds(orow0, _OPC)], wsems[par]
                ).wait()

            extract(j, gbufs[par], obufs[par])

            @pl.when(j + 2 < _NCH)
            def _next():
                fire(j + 2, gbufs[par], gsems[par])

            pltpu.make_async_copy(
                obufs[par],
                out_hbm.at[pl.ds(orow0 + j * _OPC, _OPC)],
                wsems[par],
            ).start()
        return 0

    lax.fori_loop(0, _NCH // 2, step, 0)

    for par in range(2):
        pltpu.make_async_copy(
            obufs[par], out_hbm.at[pl.ds(orow0, _OPC)], wsems[par]
        ).wait()


@jax.jit
def _run(x2d, table3):
    p1 = pl.kernel(
        _p1_body,
        mesh=plsc.VectorSubcoreMesh(core_axis_name="c", subcore_axis_name="s"),
        out_type=jax.ShapeDtypeStruct((_NQ, 128), jnp.float32),
        scratch_types=[
            pltpu.VMEM((_TB, 8, EMBED_DIM), jnp.float32),
            pltpu.VMEM((_TB, 8, EMBED_DIM), jnp.float32),
            pltpu.VMEM((2 * _TB, 128), jnp.float32),
            pltpu.VMEM((2 * _TB, 128), jnp.float32),
            pltpu.SemaphoreType.DMA,
            pltpu.SemaphoreType.DMA,
            pltpu.SemaphoreType.DMA,
            pltpu.SemaphoreType.DMA,
        ],
        compiler_params=pltpu.CompilerParams(
            use_tc_tiling_on_sc=True, needs_layout_passes=False
        ),
    )
    stg = p1(table3)

    p2 = pl.kernel(
        _p2_body,
        mesh=plsc.VectorSubcoreMesh(core_axis_name="c", subcore_axis_name="s"),
        out_type=jax.ShapeDtypeStruct((_ROWS * EMBED_DIM // 128, 128), jnp.float32),
        scratch_types=[
            pltpu.VMEM((_RPW // 128, 128), jnp.int32),
            pltpu.VMEM((_RPW // 128, 128), jnp.int32),
            pltpu.VMEM((_CH, 128), jnp.float32),
            pltpu.VMEM((_CH, 128), jnp.float32),
            pltpu.VMEM((_OPC, 128), jnp.float32),
            pltpu.VMEM((_OPC, 128), jnp.float32),
            pltpu.SemaphoreType.DMA,
            pltpu.SemaphoreType.DMA,
            pltpu.SemaphoreType.DMA,
            pltpu.SemaphoreType.DMA,
        ],
        compiler_params=pltpu.CompilerParams(
            use_tc_tiling_on_sc=False, needs_layout_passes=False
        ),
    )
    return p2(x2d, stg)


def kernel(x, tables):
    x2d = x.astype(jnp.int32).reshape(_ROWS // 128, 128)
    table3 = tables.reshape(_NTILE, 8, EMBED_DIM)
    out = _run(x2d, table3)
    return out.reshape(BATCH, FEATURE_NUM, EMBED_DIM)


# speedup vs baseline: 1.8599x; 1.4094x over previous
import functools

import jax
import jax.numpy as jnp
from jax import lax
from jax.experimental import pallas as pl
from jax.experimental.pallas import tpu as pltpu
from jax.experimental.pallas import tpu_sc as plsc

FEATURE_NUM = 26
VOCAB = 100000
EMBED_DIM = 32
BATCH = 16384

_L = 16
_NC = 2
_NS = 16
_NW = _NC * _NS

_ROWS = BATCH * FEATURE_NUM
_NTILE = FEATURE_NUM * VOCAB // 8
_NQ = FEATURE_NUM * VOCAB // 4

_QUADS = _NTILE // 4
_QPW = _QUADS // _NW
_TB = 48
_QPB = _TB // 4
_P1_BLOCKS = 212

_RPW = _ROWS // _NW
_CH = 128
_NCH = _RPW // _CH
_CPS = 4
_SUPER = _CPS * _CH
_NSUPER = _RPW // _SUPER


def _p1_body(table_hbm, stg_hbm, vb0, vb1, vp0, vp1, rsem0, rsem1, wsem0, wsem1):
    wid = lax.axis_index("s") * _NC + lax.axis_index("c")
    nq = _QPW + jnp.where(wid < 2, 1, 0)
    baseq = wid * _QPW + lax.min(wid, 2)

    def t0_of(k):
        return 4 * (baseq + lax.min(k * _QPB, nq - _QPB))

    def fire_read(k, vb, rsem):
        pltpu.make_async_copy(
            table_hbm.at[pl.ds(t0_of(k), _TB)], vb, rsem
        ).start()

    def repack(vb, vp):
        def one(q, _):
            for qq in range(4):
                for h in range(4):
                    for t in range(2):
                        vp[4 * q + qq, pl.ds(h * EMBED_DIM + t * _L, _L)] = (
                            vb[2 * q + (4 * qq + h) // 8, (4 * qq + h) % 8,
                               pl.ds(t * _L, _L)]
                        )
            return 0

        lax.fori_loop(0, _TB * 8 // 16, one, 0)

    vbs = (vb0, vb1)
    vps = (vp0, vp1)
    rsems = (rsem0, rsem1)
    wsems = (wsem0, wsem1)

    fire_read(0, vb0, rsem0)
    fire_read(1, vb1, rsem1)

    def step(kk, _):
        for par in range(2):
            k = 2 * kk + par
            pltpu.make_async_copy(
                table_hbm.at[pl.ds(t0_of(k), _TB)], vbs[par], rsems[par]
            ).wait()

            @pl.when(kk >= 1)
            def _wait_wb():
                pltpu.make_async_copy(
                    vps[par], stg_hbm.at[pl.ds(0, 2 * _TB)], wsems[par]
                ).wait()

            repack(vbs[par], vps[par])

            @pl.when(k + 2 < _P1_BLOCKS)
            def _next_read():
                fire_read(k + 2, vbs[par], rsems[par])

            pltpu.make_async_copy(
                vps[par], stg_hbm.at[pl.ds(2 * t0_of(k), 2 * _TB)], wsems[par]
            ).start()
        return 0

    lax.fori_loop(0, _P1_BLOCKS // 2, step, 0)

    for par in range(2):
        pltpu.make_async_copy(
            vps[par], stg_hbm.at[pl.ds(0, 2 * _TB)], wsems[par]
        ).wait()


def _p2_body(x_hbm, stg_hbm, out_hbm, idx_v, rows_v, gsem, wsem):
    wid = lax.axis_index("s") * _NC + lax.axis_index("c")
    base_chunk = wid * _NCH
    base_row = wid * _RPW

    pltpu.sync_copy(x_hbm.at[pl.ds(base_chunk, _NCH)], idx_v)

    lanes = lax.iota(jnp.int32, _L)

    def compute_chunk(j, _):
        for t in range(_CH // _L):
            r = j * _CH + t * _L + lanes
            f = lax.rem(r, FEATURE_NUM)
            idx_v[j, pl.ds(t * _L, _L)] = (
                idx_v[j, pl.ds(t * _L, _L)] + f * VOCAB
            )
        return 0

    lax.fori_loop(0, _NCH, compute_chunk, 0)

    def fire_super(ss, buf):
        for k in range(_CPS):
            pltpu.make_async_copy(
                stg_hbm.at[idx_v.at[ss * _CPS + k]],
                rows_v.at[buf, pl.ds(k * _CH, _CH)],
                gsem,
            ).start()

    fire_super(0, 0)

    def step(ss, _):
        s = lax.rem(ss, 2)
        s2 = lax.rem(ss + 1, 2)

        @pl.when(ss >= 1)
        def _wait_prev_writeback():
            pltpu.make_async_copy(
                rows_v.at[s2], out_hbm.at[pl.ds(base_row, _SUPER)], wsem
            ).wait()

        @pl.when(ss + 1 < _NSUPER)
        def _fire_next():
            fire_super(ss + 1, s2)

        for k in range(_CPS):
            pltpu.make_async_copy(
                stg_hbm.at[idx_v.at[ss * _CPS + k]],
                rows_v.at[s, pl.ds(k * _CH, _CH)],
                gsem,
            ).wait()

        pltpu.make_async_copy(
            rows_v.at[s], out_hbm.at[pl.ds(base_row + ss * _SUPER, _SUPER)], wsem
        ).start()
        return 0

    lax.fori_loop(0, _NSUPER, step, 0)

    pltpu.make_async_copy(
        rows_v.at[(_NSUPER - 1) % 2],
        out_hbm.at[pl.ds(base_row, _SUPER)],
        wsem,
    ).wait()


@jax.jit
def _run(x2d, table3):
    p1 = pl.kernel(
        _p1_body,
        mesh=plsc.VectorSubcoreMesh(core_axis_name="c", subcore_axis_name="s"),
        out_type=jax.ShapeDtypeStruct((_NQ, 128), jnp.float32),
        scratch_types=[
            pltpu.VMEM((_TB, 8, EMBED_DIM), jnp.float32),
            pltpu.VMEM((_TB, 8, EMBED_DIM), jnp.float32),
            pltpu.VMEM((2 * _TB, 128), jnp.float32),
            pltpu.VMEM((2 * _TB, 128), jnp.float32),
            pltpu.SemaphoreType.DMA,
            pltpu.SemaphoreType.DMA,
            pltpu.SemaphoreType.DMA,
            pltpu.SemaphoreType.DMA,
        ],
        compiler_params=pltpu.CompilerParams(
            use_tc_tiling_on_sc=True, needs_layout_passes=False
        ),
    )
    stg = p1(table3)
    stg2d = stg.reshape(FEATURE_NUM * VOCAB, EMBED_DIM)

    p2 = pl.kernel(
        _p2_body,
        mesh=plsc.VectorSubcoreMesh(core_axis_name="c", subcore_axis_name="s"),
        out_type=jax.ShapeDtypeStruct((_ROWS, EMBED_DIM), jnp.float32),
        scratch_types=[
            pltpu.VMEM((_NCH, _CH), jnp.int32),
            pltpu.VMEM((2, _SUPER, EMBED_DIM), jnp.float32),
            pltpu.SemaphoreType.DMA,
            pltpu.SemaphoreType.DMA,
        ],
        compiler_params=pltpu.CompilerParams(use_tc_tiling_on_sc=False),
    )
    return p2(x2d, stg2d)


def kernel(x, tables):
    x2d = x.astype(jnp.int32).reshape(_ROWS // _CH, _CH)
    table3 = tables.reshape(_NTILE, 8, EMBED_DIM)
    out = _run(x2d, table3)
    return out.reshape(BATCH, FEATURE_NUM, EMBED_DIM)
